# baseline (speedup 1.0000x reference)
import jax
import jax.numpy as jnp
from jax import lax
from jax.experimental import pallas as pl
from jax.experimental.pallas import tpu as pltpu
from jax.experimental.pallas import tpu_sc as plsc

B = 16384
F = 26
V = 100000
D = 128
NUM = 13

NW = 32
CHUNK = 128

NCH = 4
BC = B // NCH
ROWS_C = F * BC
ROWS_W = ROWS_C // NW
NCHUNK = ROWS_W // CHUNK
K = 2
NGROUP = NCHUNK // K

assert ROWS_W % CHUNK == 0 and NCHUNK % K == 0


def _sc_gather_body(tab_hbm, idx_hbm, out_hbm, idx_v, bufs, gsem, wsem):
    wid = lax.axis_index("s") * 2 + lax.axis_index("c")
    base = wid * ROWS_W
    pltpu.sync_copy(idx_hbm.at[wid], idx_v)

    def group(g, carry):
        j0 = g * K

        @pl.when(g > 0)
        def _():
            for b in range(K):
                pltpu.make_async_copy(
                    bufs.at[b], out_hbm.at[pl.ds(base, CHUNK)], wsem
                ).wait()

        for b in range(K):
            pltpu.make_async_copy(
                tab_hbm.at[idx_v.at[j0 + b]], bufs.at[b], gsem
            ).start()
        for b in range(K):
            pltpu.make_async_copy(
                tab_hbm.at[idx_v.at[j0 + b]], bufs.at[b], gsem
            ).wait()
        for b in range(K):
            pltpu.make_async_copy(
                bufs.at[b],
                out_hbm.at[pl.ds(base + (j0 + b) * CHUNK, CHUNK)],
                wsem,
            ).start()
        return carry

    lax.fori_loop(0, NGROUP, group, 0)
    for b in range(K):
        pltpu.make_async_copy(
            bufs.at[b], out_hbm.at[pl.ds(base, CHUNK)], wsem
        ).wait()


_sc_gather = pl.kernel(
    _sc_gather_body,
    out_type=jax.ShapeDtypeStruct((ROWS_C, D), jnp.float32),
    mesh=plsc.VectorSubcoreMesh(core_axis_name="c", subcore_axis_name="s"),
    scratch_types=[
        pltpu.VMEM((NCHUNK, CHUNK), jnp.int32),
        pltpu.VMEM((K, CHUNK, D), jnp.float32),
        pltpu.SemaphoreType.DMA,
        pltpu.SemaphoreType.DMA,
    ],
)


BLK = 256


def _tc_body(emb_ref, xnum_ref, wn_ref, bn_ref, w1f_ref, w1n_ref, b1_ref,
             gamma_ref, beta_ref, out_ref):
    cdims = (((1,), (1,)), ((), ()))
    hp = lax.dot_general(emb_ref[...], w1f_ref[...],
                         (((2,), (2,)), ((0,), (0,))),
                         preferred_element_type=jnp.float32)
    h = jnp.sum(hp, axis=0)
    num = lax.dot_general(xnum_ref[...], wn_ref[...], cdims,
                          preferred_element_type=jnp.float32) + bn_ref[...]
    h = h + lax.dot_general(num, w1n_ref[...], cdims,
                            preferred_element_type=jnp.float32) + b1_ref[...]
    mu = jnp.mean(h, axis=-1, keepdims=True)
    d = h - mu
    var = jnp.mean(d * d, axis=-1, keepdims=True)
    hn = d * lax.rsqrt(var + 1e-5) * gamma_ref[...] + beta_ref[...]
    out_ref[...] = jnp.maximum(hn, 0.0)


def _tc_fused(emb3, x_num, Wn, bn, W1f, W1n, b1, gamma, beta):
    full = lambda s: pl.BlockSpec(s, lambda i: tuple(0 for _ in s))
    return pl.pallas_call(
        _tc_body,
        grid=(BC // BLK,),
        in_specs=[
            pl.BlockSpec((F, BLK, D), lambda i: (0, i, 0)),
            pl.BlockSpec((BLK, NUM), lambda i: (i, 0)),
            full((D, NUM)),
            full((1, D)),
            full((F, D, D)),
            full((D, D)),
            full((1, D)),
            full((1, D)),
            full((1, D)),
        ],
        out_specs=pl.BlockSpec((BLK, D), lambda i: (i, 0)),
        out_shape=jax.ShapeDtypeStruct((BC, D), jnp.float32),
    )(emb3, x_num, Wn, bn, W1f, W1n, b1, gamma, beta)


def kernel(x_cat, x_num, tables, Wn, bn, W1, b1, gamma, beta):
    idx_full = (x_cat.astype(jnp.int32).T
                + (jnp.arange(F, dtype=jnp.int32) * V)[:, None])
    tab = tables.reshape(F * V, D)
    bn2, b12 = bn.reshape(1, D), b1.reshape(1, D)
    g2, be2 = gamma.reshape(1, D), beta.reshape(1, D)
    W1f = W1[:, :F * D].reshape(D, F, D).transpose(1, 0, 2)
    W1n = W1[:, F * D:]
    outs = []
    for c in range(NCH):
        idx_c = idx_full[:, c * BC:(c + 1) * BC].reshape(NW, NCHUNK, CHUNK)
        emb = _sc_gather(tab, idx_c)
        emb3 = emb.reshape(F, BC, D)
        outs.append(_tc_fused(
            emb3, x_num[c * BC:(c + 1) * BC], Wn, bn2, W1f, W1n, b12,
            g2, be2,
        ))
    return jnp.concatenate(outs, axis=0)


# speedup vs baseline: 1.0347x; 1.0347x over previous
import jax
import jax.numpy as jnp
from jax import lax
from jax.experimental import pallas as pl
from jax.experimental.pallas import tpu as pltpu
from jax.experimental.pallas import tpu_sc as plsc

B = 16384
F = 26
V = 100000
D = 128
NUM = 13

NW = 32
CHUNK = 128

NCH = 2
BC = B // NCH
ROWS_C = F * BC
ROWS_W = ROWS_C // NW
NCHUNK = ROWS_W // CHUNK
K = 4
NGROUP = NCHUNK // K

assert ROWS_W % CHUNK == 0 and NCHUNK % K == 0


def _sc_gather_body(tab_hbm, idx_hbm, out_hbm, idx_v, bufs, gsem, wsem):
    wid = lax.axis_index("s") * 2 + lax.axis_index("c")
    base = wid * ROWS_W
    pltpu.sync_copy(idx_hbm.at[wid], idx_v)

    def group(g, carry):
        j0 = g * K

        @pl.when(g > 0)
        def _():
            for b in range(K):
                pltpu.make_async_copy(
                    bufs.at[b], out_hbm.at[pl.ds(base, CHUNK)], wsem
                ).wait()

        for b in range(K):
            pltpu.make_async_copy(
                tab_hbm.at[idx_v.at[j0 + b]], bufs.at[b], gsem
            ).start()
        for b in range(K):
            pltpu.make_async_copy(
                tab_hbm.at[idx_v.at[j0 + b]], bufs.at[b], gsem
            ).wait()
        for b in range(K):
            pltpu.make_async_copy(
                bufs.at[b],
                out_hbm.at[pl.ds(base + (j0 + b) * CHUNK, CHUNK)],
                wsem,
            ).start()
        return carry

    lax.fori_loop(0, NGROUP, group, 0)
    for b in range(K):
        pltpu.make_async_copy(
            bufs.at[b], out_hbm.at[pl.ds(base, CHUNK)], wsem
        ).wait()


_sc_gather = pl.kernel(
    _sc_gather_body,
    out_type=jax.ShapeDtypeStruct((ROWS_C, D), jnp.float32),
    mesh=plsc.VectorSubcoreMesh(core_axis_name="c", subcore_axis_name="s"),
    scratch_types=[
        pltpu.VMEM((NCHUNK, CHUNK), jnp.int32),
        pltpu.VMEM((K, CHUNK, D), jnp.float32),
        pltpu.SemaphoreType.DMA,
        pltpu.SemaphoreType.DMA,
    ],
)


BLK = 256


def _tc_body(emb_ref, xnum_ref, wn_ref, bn_ref, w1f_ref, w1n_ref, b1_ref,
             gamma_ref, beta_ref, out_ref):
    cdims = (((1,), (1,)), ((), ()))
    hp = lax.dot_general(emb_ref[...], w1f_ref[...],
                         (((2,), (2,)), ((0,), (0,))),
                         preferred_element_type=jnp.float32)
    h = jnp.sum(hp, axis=0)
    num = lax.dot_general(xnum_ref[...], wn_ref[...], cdims,
                          preferred_element_type=jnp.float32) + bn_ref[...]
    h = h + lax.dot_general(num, w1n_ref[...], cdims,
                            preferred_element_type=jnp.float32) + b1_ref[...]
    mu = jnp.mean(h, axis=-1, keepdims=True)
    d = h - mu
    var = jnp.mean(d * d, axis=-1, keepdims=True)
    hn = d * lax.rsqrt(var + 1e-5) * gamma_ref[...] + beta_ref[...]
    out_ref[...] = jnp.maximum(hn, 0.0)


def _tc_fused(emb3, x_num, Wn, bn, W1f, W1n, b1, gamma, beta):
    full = lambda s: pl.BlockSpec(s, lambda i: tuple(0 for _ in s))
    return pl.pallas_call(
        _tc_body,
        grid=(BC // BLK,),
        in_specs=[
            pl.BlockSpec((F, BLK, D), lambda i: (0, i, 0)),
            pl.BlockSpec((BLK, NUM), lambda i: (i, 0)),
            full((D, NUM)),
            full((1, D)),
            full((F, D, D)),
            full((D, D)),
            full((1, D)),
            full((1, D)),
            full((1, D)),
        ],
        out_specs=pl.BlockSpec((BLK, D), lambda i: (i, 0)),
        out_shape=jax.ShapeDtypeStruct((BC, D), jnp.float32),
    )(emb3, x_num, Wn, bn, W1f, W1n, b1, gamma, beta)


def kernel(x_cat, x_num, tables, Wn, bn, W1, b1, gamma, beta):
    idx_full = (x_cat.astype(jnp.int32).T
                + (jnp.arange(F, dtype=jnp.int32) * V)[:, None])
    tab = tables.reshape(F * V, D)
    bn2, b12 = bn.reshape(1, D), b1.reshape(1, D)
    g2, be2 = gamma.reshape(1, D), beta.reshape(1, D)
    W1f = W1[:, :F * D].reshape(D, F, D).transpose(1, 0, 2)
    W1n = W1[:, F * D:]
    outs = []
    for c in range(NCH):
        idx_c = idx_full[:, c * BC:(c + 1) * BC].reshape(NW, NCHUNK, CHUNK)
        emb = _sc_gather(tab, idx_c)
        emb3 = emb.reshape(F, BC, D)
        outs.append(_tc_fused(
            emb3, x_num[c * BC:(c + 1) * BC], Wn, bn2, W1f, W1n, b12,
            g2, be2,
        ))
    return jnp.concatenate(outs, axis=0)
